# Initial kernel scaffold; baseline (speedup 1.0000x reference)
import jax
import jax.numpy as jnp
from jax.experimental import pallas as pl


def kernel(x_bytes, emb_weight):
    raise NotImplementedError("write your pallas kernel here")


# speedup vs baseline: 9.8049x; 9.8049x over previous
import functools

import jax
import jax.numpy as jnp
from jax import lax
from jax.experimental import pallas as pl
from jax.experimental.pallas import tpu as pltpu
from jax.experimental.pallas import tpu_sc as plsc

VOCAB = 100000
DIM = 64
B = 1024
T = 200
NGRAM_COUNT = 4 * T - 10
IDS_PAD = 800
NCHUNK = 10
CHUNK = 80

NC = 2
NS = 16
NW = NC * NS
ROWS_PER_W = B // NW


def _ids_body(x_ref, out_ref):
    xp = x_ref[...] + 1
    t2 = xp[:, 0:199] * 257 + xp[:, 1:200]
    i2 = t2
    i3 = (i2[:, 0:198] * 257 + xp[:, 2:200]) % VOCAB
    i4 = (i3[:, 0:197] * 257 + xp[:, 3:200]) % VOCAB
    i5 = (i4[:, 0:196] * 257 + xp[:, 4:200]) % VOCAB
    pad = jnp.zeros((B, IDS_PAD - NGRAM_COUNT), dtype=jnp.int32)
    out_ref[...] = jnp.concatenate([i2, i3, i4, i5, pad], axis=1)


def _compute_ids(x32):
    return pl.pallas_call(
        _ids_body,
        out_shape=jax.ShapeDtypeStruct((B, IDS_PAD), jnp.int32),
    )(x32)


def _sc_body(ids_hbm, table_hbm, out_hbm, idx_v, buf_v, acc_v, sem):
    wid = lax.axis_index("s") * NC + lax.axis_index("c")
    base = wid * ROWS_PER_W

    def row_body(r, _):
        row = base + r
        pltpu.sync_copy(ids_hbm.at[row], idx_v)
        copies = []
        for j in range(NCHUNK):
            copies.append(
                pltpu.async_copy(
                    table_hbm.at[idx_v.at[jnp.int32(j)]],
                    buf_v.at[pl.ds(jnp.int32(j * CHUNK), CHUNK)],
                    sem,
                )
            )
        for c in copies:
            c.wait()

        def acc_body(i, carry):
            a0, a1, a2, a3 = carry
            a0 = a0 + buf_v[i, pl.ds(0, 16)]
            a1 = a1 + buf_v[i, pl.ds(16, 16)]
            a2 = a2 + buf_v[i, pl.ds(32, 16)]
            a3 = a3 + buf_v[i, pl.ds(48, 16)]
            return (a0, a1, a2, a3)

        z = jnp.zeros((16,), jnp.float32)
        a0, a1, a2, a3 = lax.fori_loop(
            jnp.int32(0), jnp.int32(NGRAM_COUNT), acc_body, (z, z, z, z))
        acc_v[r, pl.ds(0, 16)] = a0
        acc_v[r, pl.ds(16, 16)] = a1
        acc_v[r, pl.ds(32, 16)] = a2
        acc_v[r, pl.ds(48, 16)] = a3
        return _

    lax.fori_loop(jnp.int32(0), jnp.int32(ROWS_PER_W), row_body, None)
    pltpu.sync_copy(acc_v, out_hbm.at[pl.ds(base, ROWS_PER_W)])


@functools.cache
def _gather_sums_fn():
    return pl.kernel(
        _sc_body,
        out_type=jax.ShapeDtypeStruct((B, DIM), jnp.float32),
        mesh=plsc.VectorSubcoreMesh(core_axis_name="c", subcore_axis_name="s"),
        scratch_types=[
            pltpu.VMEM((NCHUNK, CHUNK), jnp.int32),
            pltpu.VMEM((IDS_PAD, DIM), jnp.float32),
            pltpu.VMEM((ROWS_PER_W, DIM), jnp.float32),
            pltpu.SemaphoreType.DMA,
        ],
        compiler_params=pltpu.CompilerParams(use_tc_tiling_on_sc=False),
    )


def _norm_body(s_ref, out_ref):
    p = s_ref[...] * (1.0 / NGRAM_COUNT)
    n2 = jnp.sum(p * p, axis=1, keepdims=True)
    norm = jnp.maximum(jnp.sqrt(n2), 1e-12)
    out_ref[...] = p / norm


def _normalize(sums):
    return pl.pallas_call(
        _norm_body,
        out_shape=jax.ShapeDtypeStruct((B, DIM), jnp.float32),
    )(sums)


def kernel(x_bytes, emb_weight):
    x32 = x_bytes.astype(jnp.int32)
    ids = _compute_ids(x32)
    ids3 = ids.reshape(B, NCHUNK, CHUNK)
    sums = _gather_sums_fn()(ids3, emb_weight)
    return _normalize(sums)
